# baseline (speedup 1.0000x reference)
import functools

import jax
import jax.numpy as jnp
from jax import lax
from jax.experimental import pallas as pl
from jax.experimental.pallas import tpu as pltpu
from jax.experimental.pallas import tpu_sc as plsc

_VOCAB = 100000
_D = 64
_B = 1024
_CTX = 20

_NC = 2
_NS = 16
_NW = _NC * _NS
_BPW = _B // _NW
_IPW = _BPW * _CTX
_BPC = 4
_ICHUNK = _BPC * _CTX
_NCHUNK = _IPW // _ICHUNK

_mesh = plsc.VectorSubcoreMesh(core_axis_name="c", subcore_axis_name="s")


@functools.partial(
    pl.kernel,
    out_type=jax.ShapeDtypeStruct((_B, _D), jnp.float32),
    mesh=_mesh,
    scratch_types=[
        pltpu.VMEM((_NCHUNK, _ICHUNK), jnp.int32),
        pltpu.VMEM((2, _ICHUNK, 2 * _D), jnp.float32),
        pltpu.VMEM((_IPW, 16), jnp.float32),
        pltpu.VMEM((_BPW, _D), jnp.float32),
        pltpu.SemaphoreType.DMA,
        pltpu.SemaphoreType.DMA,
    ],
)
def _sc_hidden(table_hbm, idx_hbm, par_hbm, out_hbm, idx_v, rows_v, par_v,
               hid_v, sem0, sem1):
    wid = lax.axis_index("s") * _NC + lax.axis_index("c")
    pltpu.sync_copy(idx_hbm.at[wid], idx_v)
    pltpu.sync_copy(par_hbm.at[wid], par_v)

    sems = (sem0, sem1)
    inv = jnp.float32(1.0 / _CTX)
    ngrp = _D // 16

    def gather(k):
        return pltpu.async_copy(
            table_hbm.at[idx_v.at[k]], rows_v.at[k % 2], sems[k % 2]
        )

    pending = gather(0)
    for k in range(_NCHUNK):
        if k + 1 < _NCHUNK:
            nxt = gather(k + 1)
        pending.wait()

        def body(bl, carry, _k=k):
            base = bl * _CTX
            acc = [jnp.zeros((16,), jnp.float32)] * ngrp
            for c in range(_CTX):
                r = base + c
                p = par_v[_k * _ICHUNK + r]
                for d in range(ngrp):
                    a0 = rows_v[_k % 2, r, pl.ds(d * 16, 16)]
                    a1 = rows_v[_k % 2, r, pl.ds(_D + d * 16, 16)]
                    acc[d] = acc[d] + (a0 + p * (a1 - a0))
            for d in range(ngrp):
                hid_v[_k * _BPC + bl, pl.ds(d * 16, 16)] = acc[d] * inv
            return carry

        lax.fori_loop(0, _BPC, body, 0)
        if k + 1 < _NCHUNK:
            pending = nxt

    pltpu.sync_copy(hid_v, out_hbm.at[pl.ds(wid * _BPW, _BPW)])


_TV = 512


def _mm_body(h_ref, w_ref, b_ref, o_ref):
    o_ref[...] = (
        lax.dot_general(
            h_ref[...],
            w_ref[...],
            (((1,), (1,)), ((), ())),
            preferred_element_type=jnp.float32,
        )
        + b_ref[...]
    )


def _tc_linear(hidden, lin_w, lin_b2d):
    grid = (pl.cdiv(_VOCAB, _TV),)
    return pl.pallas_call(
        _mm_body,
        grid=grid,
        in_specs=[
            pl.BlockSpec((_B, _D), lambda i: (0, 0)),
            pl.BlockSpec((_TV, _D), lambda i: (i, 0)),
            pl.BlockSpec((1, _TV), lambda i: (0, i)),
        ],
        out_specs=pl.BlockSpec((_B, _TV), lambda i: (0, i)),
        out_shape=jax.ShapeDtypeStruct((_B, _VOCAB), jnp.float32),
        compiler_params=pltpu.CompilerParams(
            dimension_semantics=("arbitrary",),
        ),
    )(hidden, lin_w, lin_b2d)


@jax.jit
def kernel(context_idxs, emb_table, lin_w, lin_b):
    idx = context_idxs.astype(jnp.int32)
    idx_half = (idx >> 1).reshape(_NW, _NCHUNK, _ICHUNK)
    par = (idx & 1).astype(jnp.float32).reshape(_NW, _IPW, 1)
    par = jnp.broadcast_to(par, (_NW, _IPW, 16))
    table2 = emb_table.reshape(_VOCAB // 2, 2 * _D)
    hidden = _sc_hidden(table2, idx_half, par)
    return _tc_linear(hidden, lin_w, lin_b.reshape(1, _VOCAB))


# speedup vs baseline: 1.0335x; 1.0335x over previous
import functools

import jax
import jax.numpy as jnp
from jax import lax
from jax.experimental import pallas as pl
from jax.experimental.pallas import tpu as pltpu
from jax.experimental.pallas import tpu_sc as plsc

_VOCAB = 100000
_D = 64
_B = 1024
_CTX = 20

_NC = 2
_NS = 16
_NW = _NC * _NS
_BPW = _B // _NW
_IPW = _BPW * _CTX
_BPC = 4
_ICHUNK = _BPC * _CTX
_NCHUNK = _IPW // _ICHUNK

_mesh = plsc.VectorSubcoreMesh(core_axis_name="c", subcore_axis_name="s")


@functools.partial(
    pl.kernel,
    out_type=jax.ShapeDtypeStruct((_B, _D), jnp.float32),
    mesh=_mesh,
    scratch_types=[
        pltpu.VMEM((_NCHUNK, _ICHUNK), jnp.int32),
        pltpu.VMEM((2, _ICHUNK, 2 * _D), jnp.float32),
        pltpu.VMEM((_IPW, 16), jnp.float32),
        pltpu.VMEM((_BPW, _D), jnp.float32),
        pltpu.SemaphoreType.DMA,
        pltpu.SemaphoreType.DMA,
    ],
)
def _sc_hidden(table_hbm, idx_hbm, par_hbm, out_hbm, idx_v, rows_v, par_v,
               hid_v, sem0, sem1):
    wid = lax.axis_index("s") * _NC + lax.axis_index("c")
    pltpu.sync_copy(idx_hbm.at[wid], idx_v)
    pltpu.sync_copy(par_hbm.at[wid], par_v)

    sems = (sem0, sem1)
    inv = jnp.float32(1.0 / _CTX)
    ngrp = _D // 16

    def gather(k):
        return pltpu.async_copy(
            table_hbm.at[idx_v.at[k]], rows_v.at[k % 2], sems[k % 2]
        )

    pending = gather(0)
    for k in range(_NCHUNK):
        if k + 1 < _NCHUNK:
            nxt = gather(k + 1)
        pending.wait()

        def body(bl, carry, _k=k):
            base = bl * _CTX
            acc = [jnp.zeros((16,), jnp.float32)] * ngrp
            for c in range(_CTX):
                r = base + c
                p = par_v[_k * _ICHUNK + r]
                for d in range(ngrp):
                    a0 = rows_v[_k % 2, r, pl.ds(d * 16, 16)]
                    a1 = rows_v[_k % 2, r, pl.ds(_D + d * 16, 16)]
                    acc[d] = acc[d] + (a0 + p * (a1 - a0))
            for d in range(ngrp):
                hid_v[_k * _BPC + bl, pl.ds(d * 16, 16)] = acc[d] * inv
            return carry

        lax.fori_loop(0, _BPC, body, 0)
        if k + 1 < _NCHUNK:
            pending = nxt

    pltpu.sync_copy(hid_v, out_hbm.at[pl.ds(wid * _BPW, _BPW)])


_TV = 512


def _mm_body(h_ref, w_ref, b_ref, o_ref):
    o_ref[...] = (
        lax.dot_general(
            h_ref[...],
            w_ref[...],
            (((1,), (1,)), ((), ())),
            preferred_element_type=jnp.float32,
        )
        + b_ref[...]
    )


def _tc_linear(hidden, lin_w, lin_b2d):
    grid = (pl.cdiv(_VOCAB, _TV),)
    return pl.pallas_call(
        _mm_body,
        grid=grid,
        in_specs=[
            pl.BlockSpec((_B, _D), lambda i: (0, 0)),
            pl.BlockSpec((_TV, _D), lambda i: (i, 0)),
            pl.BlockSpec((1, _TV), lambda i: (0, i)),
        ],
        out_specs=pl.BlockSpec((_B, _TV), lambda i: (0, i)),
        out_shape=jax.ShapeDtypeStruct((_B, _VOCAB), jnp.float32),
        compiler_params=pltpu.CompilerParams(
            dimension_semantics=("arbitrary",),
        ),
    )(hidden, lin_w, lin_b2d)


@jax.jit
def kernel(context_idxs, emb_table, lin_w, lin_b):
    hidden = jnp.mean(jnp.take(emb_table, context_idxs, axis=0), axis=1)
    return _tc_linear(hidden, lin_w, lin_b.reshape(1, _VOCAB))


# speedup vs baseline: 1.1692x; 1.1312x over previous
import functools

import jax
import jax.numpy as jnp
from jax import lax
from jax.experimental import pallas as pl
from jax.experimental.pallas import tpu as pltpu
from jax.experimental.pallas import tpu_sc as plsc

_VOCAB = 100000
_D = 64
_B = 1024
_CTX = 20

_NC = 2
_NS = 16
_NW = _NC * _NS
_BPW = _B // _NW
_IPW = _BPW * _CTX
_BPC = 4
_ICHUNK = _BPC * _CTX
_NCHUNK = _IPW // _ICHUNK

_mesh = plsc.VectorSubcoreMesh(core_axis_name="c", subcore_axis_name="s")


@functools.partial(
    pl.kernel,
    out_type=jax.ShapeDtypeStruct((_B, _D), jnp.float32),
    mesh=_mesh,
    scratch_types=[
        pltpu.VMEM((_NCHUNK, _ICHUNK), jnp.int32),
        pltpu.VMEM((2, _ICHUNK, 2 * _D), jnp.float32),
        pltpu.VMEM((_IPW, 16), jnp.float32),
        pltpu.VMEM((_BPW, _D), jnp.float32),
        pltpu.SemaphoreType.DMA,
        pltpu.SemaphoreType.DMA,
    ],
)
def _sc_hidden(table_hbm, idx_hbm, par_hbm, out_hbm, idx_v, rows_v, par_v,
               hid_v, sem0, sem1):
    wid = lax.axis_index("s") * _NC + lax.axis_index("c")
    pltpu.sync_copy(idx_hbm.at[wid], idx_v)
    pltpu.sync_copy(par_hbm.at[wid], par_v)

    sems = (sem0, sem1)
    inv = jnp.float32(1.0 / _CTX)
    ngrp = _D // 16

    def gather(k):
        return pltpu.async_copy(
            table_hbm.at[idx_v.at[k]], rows_v.at[k % 2], sems[k % 2]
        )

    pending = gather(0)
    for k in range(_NCHUNK):
        if k + 1 < _NCHUNK:
            nxt = gather(k + 1)
        pending.wait()

        def body(bl, carry, _k=k):
            base = bl * _CTX
            acc = [jnp.zeros((16,), jnp.float32)] * ngrp
            for c in range(_CTX):
                r = base + c
                p = par_v[_k * _ICHUNK + r]
                for d in range(ngrp):
                    a0 = rows_v[_k % 2, r, pl.ds(d * 16, 16)]
                    a1 = rows_v[_k % 2, r, pl.ds(_D + d * 16, 16)]
                    acc[d] = acc[d] + (a0 + p * (a1 - a0))
            for d in range(ngrp):
                hid_v[_k * _BPC + bl, pl.ds(d * 16, 16)] = acc[d] * inv
            return carry

        lax.fori_loop(0, _BPC, body, 0)
        if k + 1 < _NCHUNK:
            pending = nxt

    pltpu.sync_copy(hid_v, out_hbm.at[pl.ds(wid * _BPW, _BPW)])


_TV = 2048


def _mm_body(h_ref, w_ref, b_ref, o_ref):
    o_ref[...] = (
        lax.dot_general(
            h_ref[...],
            w_ref[...],
            (((1,), (1,)), ((), ())),
            preferred_element_type=jnp.float32,
        )
        + b_ref[...]
    )


def _tc_linear(hidden, lin_w, lin_b2d):
    grid = (pl.cdiv(_VOCAB, _TV),)
    return pl.pallas_call(
        _mm_body,
        grid=grid,
        in_specs=[
            pl.BlockSpec((_B, _D), lambda i: (0, 0)),
            pl.BlockSpec((_TV, _D), lambda i: (i, 0)),
            pl.BlockSpec((1, _TV), lambda i: (0, i)),
        ],
        out_specs=pl.BlockSpec((_B, _TV), lambda i: (0, i)),
        out_shape=jax.ShapeDtypeStruct((_B, _VOCAB), jnp.float32),
        compiler_params=pltpu.CompilerParams(
            dimension_semantics=("arbitrary",),
        ),
    )(hidden, lin_w, lin_b2d)


@jax.jit
def kernel(context_idxs, emb_table, lin_w, lin_b):
    hidden = jnp.mean(jnp.take(emb_table, context_idxs, axis=0), axis=1)
    return _tc_linear(hidden, lin_w, lin_b.reshape(1, _VOCAB))
